# initial kernel scaffold (unmeasured)
import jax
import jax.numpy as jnp
from jax import lax
from jax.experimental import pallas as pl
from jax.experimental.pallas import tpu as pltpu

N_DEV = 4
S = 4096
D = 1024
CHUNK = S // N_DEV
DH = 128
SCALE = 0.08838834764831843
EPS = 1e-5


def _fused_body(
    ao_ref, wo_ref, x0_ref, ga_ref, sm_ref, shm_ref, gm_ref, w1_ref, w2_ref,
    out_ref, p_ref, x1_ref, recv_ref, send_sems, recv_sems,
):
    my = lax.axis_index("i")
    left = lax.rem(my + N_DEV - 1, N_DEV)
    right = lax.rem(my + 1, N_DEV)

    barrier = pltpu.get_barrier_semaphore()
    pl.semaphore_signal(barrier, inc=1, device_id=(left,),
                        device_id_type=pl.DeviceIdType.MESH)
    pl.semaphore_signal(barrier, inc=1, device_id=(right,),
                        device_id_type=pl.DeviceIdType.MESH)
    pl.semaphore_wait(barrier, 2)

    def all_reduce(dst_ref, make_final, sem_base, recv_base):
        for h in range(N_DEV - 1):
            s_idx = lax.rem(my + ((-h) % N_DEV), N_DEV)
            r_idx = lax.rem(my + ((-h - 1) % N_DEV), N_DEV)
            rdma = pltpu.make_async_remote_copy(
                src_ref=p_ref.at[s_idx],
                dst_ref=recv_ref.at[recv_base + h],
                send_sem=send_sems.at[sem_base + h],
                recv_sem=recv_sems.at[sem_base + h],
                device_id=(right,),
                device_id_type=pl.DeviceIdType.MESH,
            )
            rdma.start()
            rdma.wait()
            p_ref[r_idx] = p_ref[r_idx] + recv_ref[recv_base + h]

        g = lax.rem(my + 1, N_DEV)
        make_final(g)

        for h in range(N_DEV - 1):
            c = lax.rem(my + ((1 - h) % N_DEV), N_DEV)
            rdma = pltpu.make_async_remote_copy(
                src_ref=dst_ref.at[c],
                dst_ref=dst_ref.at[c],
                send_sem=send_sems.at[sem_base + 3 + h],
                recv_sem=recv_sems.at[sem_base + 3 + h],
                device_id=(right,),
                device_id_type=pl.DeviceIdType.MESH,
            )
            rdma.start()
            rdma.wait()

    for c in range(N_DEV):
        p_ref[c] = jnp.dot(
            ao_ref[pl.ds(c * CHUNK, CHUNK), :], wo_ref[...],
            preferred_element_type=jnp.float32,
        ).astype(jnp.bfloat16)

    def final1(g):
        x1_ref[g] = (
            x0_ref[g].astype(jnp.float32)
            + ga_ref[...].astype(jnp.float32) * p_ref[g].astype(jnp.float32)
        ).astype(jnp.bfloat16)

    all_reduce(x1_ref, final1, sem_base=0, recv_base=0)

    for c in range(N_DEV):
        x1c = x1_ref[c].astype(jnp.float32)
        mu = jnp.mean(x1c, axis=-1, keepdims=True)
        var = jnp.mean((x1c - mu) * (x1c - mu), axis=-1, keepdims=True)
        xm = (x1c - mu) * lax.rsqrt(var + EPS)
        xm = xm * (1.0 + sm_ref[...].astype(jnp.float32)) + shm_ref[...].astype(jnp.float32)
        h1 = jnp.dot(xm.astype(jnp.bfloat16), w1_ref[...],
                     preferred_element_type=jnp.float32)
        h1 = h1 * jax.nn.sigmoid(h1)
        p_ref[c] = jnp.dot(h1.astype(jnp.bfloat16), w2_ref[...],
                           preferred_element_type=jnp.float32).astype(jnp.bfloat16)

    def final2(g):
        out_ref[g] = (
            x1_ref[g].astype(jnp.float32)
            + gm_ref[...].astype(jnp.float32) * p_ref[g].astype(jnp.float32)
        ).astype(jnp.bfloat16)

    all_reduce(out_ref, final2, sem_base=6, recv_base=3)


def _fused(ao, wo, x0c, ga, sm, shm, gm, w1, w2):
    return pl.pallas_call(
        _fused_body,
        out_shape=jax.ShapeDtypeStruct((N_DEV, CHUNK, D), jnp.bfloat16),
        in_specs=[pl.BlockSpec(memory_space=pltpu.VMEM)] * 9,
        out_specs=pl.BlockSpec(memory_space=pltpu.VMEM),
        scratch_shapes=[
            pltpu.VMEM((N_DEV, CHUNK, D), jnp.bfloat16),
            pltpu.VMEM((N_DEV, CHUNK, D), jnp.bfloat16),
            pltpu.VMEM((6, CHUNK, D), jnp.bfloat16),
            pltpu.SemaphoreType.DMA((12,)),
            pltpu.SemaphoreType.DMA((12,)),
        ],
        compiler_params=pltpu.CompilerParams(collective_id=0),
    )(ao, wo, x0c, ga, sm, shm, gm, w1, w2)


def kernel(x, Wq, Wk, Wv, Wo, t_emb, W_mod, W_ff1, W_ff2):
    f32 = jnp.float32
    bf16 = jnp.bfloat16

    x0 = x.reshape(S, D)
    mod = jnp.dot(t_emb, W_mod)
    sa, sha, ga, sm, shm, gm = jnp.split(mod, 6, axis=-1)

    mu = jnp.mean(x0, axis=-1, keepdims=True)
    var = jnp.var(x0, axis=-1, keepdims=True)
    xa = ((x0 - mu) * lax.rsqrt(var + EPS)) * (1.0 + sa) + sha
    xab = xa.astype(bf16)

    hl = Wq.shape[1] // DH
    q = jnp.dot(xab, Wq.astype(bf16), preferred_element_type=f32)
    k = jnp.dot(xab, Wk.astype(bf16), preferred_element_type=f32)
    v = jnp.dot(xab, Wv.astype(bf16), preferred_element_type=f32)
    q = q.reshape(S, hl, DH).astype(bf16)
    k = k.reshape(S, hl, DH).astype(bf16)
    v = v.reshape(S, hl, DH).astype(bf16)

    s = jnp.einsum("qhd,khd->hqk", q, k, preferred_element_type=f32) * SCALE
    m_ = jnp.max(s, axis=-1, keepdims=True)
    p = jnp.exp(s - m_)
    l_ = jnp.sum(p, axis=-1, keepdims=True)
    o = jnp.einsum("hqk,khd->qhd", (p / l_).astype(bf16), v,
                   preferred_element_type=f32)
    ao = o.reshape(S, D).astype(bf16)

    res = _fused(
        ao,
        Wo.astype(bf16),
        x0.astype(bf16).reshape(N_DEV, CHUNK, D),
        ga.astype(bf16),
        sm.astype(bf16),
        shm.astype(bf16),
        gm.astype(bf16),
        W_ff1.astype(bf16),
        W_ff2.astype(bf16),
    )
    return res.reshape(1, S, D).astype(f32)


# baseline (device time: 825163 ns/iter reference)
import jax
import jax.numpy as jnp
from jax import lax
from jax.experimental import pallas as pl
from jax.experimental.pallas import tpu as pltpu

N_DEV = 4
S = 4096
D = 1024
CHUNK = S // N_DEV
DH = 128
SCALE = 0.08838834764831843
EPS = 1e-5


def _fused_body(
    ao_ref, wo_ref, x0_ref, ga_ref, sm_ref, shm_ref, gm_ref, w1_ref, w2_ref,
    out_ref, p_ref, x1_ref, recv_ref, send_sems, recv_sems,
):
    my = lax.axis_index("i")
    left = lax.rem(my + N_DEV - 1, N_DEV)
    right = lax.rem(my + 1, N_DEV)

    barrier = pltpu.get_barrier_semaphore()
    pl.semaphore_signal(barrier, inc=1, device_id=(left,),
                        device_id_type=pl.DeviceIdType.MESH)
    pl.semaphore_signal(barrier, inc=1, device_id=(right,),
                        device_id_type=pl.DeviceIdType.MESH)
    pl.semaphore_wait(barrier, 2)

    def all_reduce(dst_ref, make_final, sem_base, recv_base):
        for h in range(N_DEV - 1):
            s_idx = lax.rem(my + ((-h) % N_DEV), N_DEV)
            r_idx = lax.rem(my + ((-h - 1) % N_DEV), N_DEV)
            rdma = pltpu.make_async_remote_copy(
                src_ref=p_ref.at[s_idx],
                dst_ref=recv_ref.at[recv_base + h],
                send_sem=send_sems.at[sem_base + h],
                recv_sem=recv_sems.at[sem_base + h],
                device_id=(right,),
                device_id_type=pl.DeviceIdType.MESH,
            )
            rdma.start()
            rdma.wait()
            p_ref[r_idx] = p_ref[r_idx] + recv_ref[recv_base + h]

        g = lax.rem(my + 1, N_DEV)
        make_final(g)

        for h in range(N_DEV - 1):
            c = lax.rem(my + ((1 - h) % N_DEV), N_DEV)
            rdma = pltpu.make_async_remote_copy(
                src_ref=dst_ref.at[c],
                dst_ref=dst_ref.at[c],
                send_sem=send_sems.at[sem_base + 3 + h],
                recv_sem=recv_sems.at[sem_base + 3 + h],
                device_id=(right,),
                device_id_type=pl.DeviceIdType.MESH,
            )
            rdma.start()
            rdma.wait()

    for c in range(N_DEV):
        p_ref[c] = jnp.dot(
            ao_ref[pl.ds(c * CHUNK, CHUNK), :], wo_ref[...],
            preferred_element_type=jnp.float32,
        ).astype(jnp.bfloat16)

    def final1(g):
        x1_ref[g] = (
            x0_ref[g].astype(jnp.float32)
            + ga_ref[...].astype(jnp.float32) * p_ref[g].astype(jnp.float32)
        ).astype(jnp.bfloat16)

    all_reduce(x1_ref, final1, sem_base=0, recv_base=0)

    for c in range(N_DEV):
        x1c = x1_ref[c].astype(jnp.float32)
        mu = jnp.mean(x1c, axis=-1, keepdims=True)
        var = jnp.mean((x1c - mu) * (x1c - mu), axis=-1, keepdims=True)
        xm = (x1c - mu) * lax.rsqrt(var + EPS)
        xm = xm * (1.0 + sm_ref[...].astype(jnp.float32)) + shm_ref[...].astype(jnp.float32)
        h1 = jnp.dot(xm.astype(jnp.bfloat16), w1_ref[...],
                     preferred_element_type=jnp.float32)
        h1 = h1 * jax.nn.sigmoid(h1)
        p_ref[c] = jnp.dot(h1.astype(jnp.bfloat16), w2_ref[...],
                           preferred_element_type=jnp.float32).astype(jnp.bfloat16)

    def final2(g):
        out_ref[g] = (
            x1_ref[g].astype(jnp.float32)
            + gm_ref[...].astype(jnp.float32) * p_ref[g].astype(jnp.float32)
        ).astype(jnp.bfloat16)

    all_reduce(out_ref, final2, sem_base=6, recv_base=3)


def _fused(ao, wo, x0c, ga, sm, shm, gm, w1, w2):
    return pl.pallas_call(
        _fused_body,
        out_shape=jax.ShapeDtypeStruct((N_DEV, CHUNK, D), jnp.bfloat16),
        in_specs=[pl.BlockSpec(memory_space=pltpu.VMEM)] * 9,
        out_specs=pl.BlockSpec(memory_space=pltpu.VMEM),
        scratch_shapes=[
            pltpu.VMEM((N_DEV, CHUNK, D), jnp.bfloat16),
            pltpu.VMEM((N_DEV, CHUNK, D), jnp.bfloat16),
            pltpu.VMEM((6, CHUNK, D), jnp.bfloat16),
            pltpu.SemaphoreType.DMA((12,)),
            pltpu.SemaphoreType.DMA((12,)),
        ],
        compiler_params=pltpu.CompilerParams(
            collective_id=0, vmem_limit_bytes=100 * 1024 * 1024,
        ),
    )(ao, wo, x0c, ga, sm, shm, gm, w1, w2)


def kernel(x, Wq, Wk, Wv, Wo, t_emb, W_mod, W_ff1, W_ff2):
    f32 = jnp.float32
    bf16 = jnp.bfloat16

    x0 = x.reshape(S, D)
    mod = jnp.dot(t_emb, W_mod)
    sa, sha, ga, sm, shm, gm = jnp.split(mod, 6, axis=-1)

    mu = jnp.mean(x0, axis=-1, keepdims=True)
    var = jnp.var(x0, axis=-1, keepdims=True)
    xa = ((x0 - mu) * lax.rsqrt(var + EPS)) * (1.0 + sa) + sha
    xab = xa.astype(bf16)

    hl = Wq.shape[1] // DH
    q = jnp.dot(xab, Wq.astype(bf16), preferred_element_type=f32)
    k = jnp.dot(xab, Wk.astype(bf16), preferred_element_type=f32)
    v = jnp.dot(xab, Wv.astype(bf16), preferred_element_type=f32)
    q = q.reshape(S, hl, DH).astype(bf16)
    k = k.reshape(S, hl, DH).astype(bf16)
    v = v.reshape(S, hl, DH).astype(bf16)

    s = jnp.einsum("qhd,khd->hqk", q, k, preferred_element_type=f32) * SCALE
    m_ = jnp.max(s, axis=-1, keepdims=True)
    p = jnp.exp(s - m_)
    l_ = jnp.sum(p, axis=-1, keepdims=True)
    o = jnp.einsum("hqk,khd->qhd", (p / l_).astype(bf16), v,
                   preferred_element_type=f32)
    ao = o.reshape(S, D).astype(bf16)

    res = _fused(
        ao,
        Wo.astype(bf16),
        x0.astype(bf16).reshape(N_DEV, CHUNK, D),
        ga.astype(bf16),
        sm.astype(bf16),
        shm.astype(bf16),
        gm.astype(bf16),
        W_ff1.astype(bf16),
        W_ff2.astype(bf16),
    )
    return res.reshape(1, S, D).astype(f32)


# device time: 821398 ns/iter; 1.0046x vs baseline; 1.0046x over previous
import jax
import jax.numpy as jnp
from jax import lax
from jax.experimental import pallas as pl
from jax.experimental.pallas import tpu as pltpu

N_DEV = 4
S = 4096
D = 1024
CHUNK = S // N_DEV
DH = 128
SCALE = 0.08838834764831843
EPS = 1e-5


def _qkv_body(
    x0_ref, wq_ref, wk_ref, wv_ref, sa_ref, sha_ref,
    q8_ref, k8_ref, v8_ref,
):
    HL = 8
    for c in range(N_DEV):
        xc = x0_ref[c].astype(jnp.float32)
        mu = jnp.mean(xc, axis=-1, keepdims=True)
        var = jnp.mean((xc - mu) * (xc - mu), axis=-1, keepdims=True)
        xa = (xc - mu) * lax.rsqrt(var + EPS)
        xa = xa * (1.0 + sa_ref[...].astype(jnp.float32)) + sha_ref[...].astype(jnp.float32)
        xab = xa.astype(jnp.bfloat16)
        qc = jnp.dot(xab, wq_ref[...], preferred_element_type=jnp.float32).astype(jnp.bfloat16)
        kc = jnp.dot(xab, wk_ref[...], preferred_element_type=jnp.float32).astype(jnp.bfloat16)
        vc = jnp.dot(xab, wv_ref[...], preferred_element_type=jnp.float32).astype(jnp.bfloat16)
        for h in range(HL):
            q8_ref[h, pl.ds(c * CHUNK, CHUNK), :] = qc[:, h * DH:(h + 1) * DH]
            k8_ref[h, pl.ds(c * CHUNK, CHUNK), :] = kc[:, h * DH:(h + 1) * DH]
            v8_ref[h, pl.ds(c * CHUNK, CHUNK), :] = vc[:, h * DH:(h + 1) * DH]


def _qkv(x0c, wq, wk, wv, sa, sha):
    shp = jax.ShapeDtypeStruct((8, S, DH), jnp.bfloat16)
    return pl.pallas_call(
        _qkv_body,
        out_shape=(shp, shp, shp),
        in_specs=[pl.BlockSpec(memory_space=pltpu.VMEM)] * 6,
        out_specs=(pl.BlockSpec(memory_space=pltpu.VMEM),) * 3,
        compiler_params=pltpu.CompilerParams(
            vmem_limit_bytes=100 * 1024 * 1024,
        ),
    )(x0c, wq, wk, wv, sa, sha)


def _attn_body(q_ref, k_ref, v_ref, o_ref):
    s = lax.dot_general(
        q_ref[0], k_ref[0], (((1,), (1,)), ((), ())),
        preferred_element_type=jnp.float32,
    ) * SCALE
    m_ = jnp.max(s, axis=-1, keepdims=True)
    p = jnp.exp(s - m_)
    l_ = jnp.sum(p, axis=-1, keepdims=True)
    o = jnp.dot((p / l_).astype(jnp.bfloat16), v_ref[0],
                preferred_element_type=jnp.float32)
    o_ref[0] = o.astype(jnp.bfloat16)


def _attn(q8, k8, v8):
    return pl.pallas_call(
        _attn_body,
        grid=(8, N_DEV),
        out_shape=jax.ShapeDtypeStruct((8, S, DH), jnp.bfloat16),
        in_specs=[
            pl.BlockSpec((1, CHUNK, DH), lambda h, c: (h, c, 0)),
            pl.BlockSpec((1, S, DH), lambda h, c: (h, 0, 0)),
            pl.BlockSpec((1, S, DH), lambda h, c: (h, 0, 0)),
        ],
        out_specs=pl.BlockSpec((1, CHUNK, DH), lambda h, c: (h, c, 0)),
        compiler_params=pltpu.CompilerParams(
            vmem_limit_bytes=100 * 1024 * 1024,
        ),
    )(q8, k8, v8)


def _fused_body(
    o8_ref, wo_ref, x0_ref, ga_ref, sm_ref, shm_ref, gm_ref, w1_ref, w2_ref,
    out_ref, p_ref, x1_ref, recv_ref, send_sems, recv_sems,
):
    my = lax.axis_index("i")
    left = lax.rem(my + N_DEV - 1, N_DEV)
    right = lax.rem(my + 1, N_DEV)

    barrier = pltpu.get_barrier_semaphore()
    pl.semaphore_signal(barrier, inc=1, device_id=(left,),
                        device_id_type=pl.DeviceIdType.MESH)
    pl.semaphore_signal(barrier, inc=1, device_id=(right,),
                        device_id_type=pl.DeviceIdType.MESH)
    pl.semaphore_wait(barrier, 2)

    def all_reduce(dst_ref, make_final, sem_base, recv_base):
        for h in range(N_DEV - 1):
            s_idx = lax.rem(my + ((-h) % N_DEV), N_DEV)
            r_idx = lax.rem(my + ((-h - 1) % N_DEV), N_DEV)
            rdma = pltpu.make_async_remote_copy(
                src_ref=p_ref.at[s_idx],
                dst_ref=recv_ref.at[recv_base + h],
                send_sem=send_sems.at[sem_base + h],
                recv_sem=recv_sems.at[sem_base + h],
                device_id=(right,),
                device_id_type=pl.DeviceIdType.MESH,
            )
            rdma.start()
            rdma.wait()
            p_ref[r_idx] = p_ref[r_idx] + recv_ref[recv_base + h]

        g = lax.rem(my + 1, N_DEV)
        make_final(g)

        for h in range(N_DEV - 1):
            c = lax.rem(my + ((1 - h) % N_DEV), N_DEV)
            rdma = pltpu.make_async_remote_copy(
                src_ref=dst_ref.at[c],
                dst_ref=dst_ref.at[c],
                send_sem=send_sems.at[sem_base + 3 + h],
                recv_sem=recv_sems.at[sem_base + 3 + h],
                device_id=(right,),
                device_id_type=pl.DeviceIdType.MESH,
            )
            rdma.start()
            rdma.wait()

    for c in range(N_DEV):
        acc = jnp.dot(o8_ref[0, pl.ds(c * CHUNK, CHUNK), :], wo_ref[0],
                      preferred_element_type=jnp.float32)
        for h in range(1, 8):
            acc = acc + jnp.dot(o8_ref[h, pl.ds(c * CHUNK, CHUNK), :], wo_ref[h],
                                preferred_element_type=jnp.float32)
        p_ref[c] = acc.astype(jnp.bfloat16)

    def final1(g):
        x1_ref[g] = (
            x0_ref[g].astype(jnp.float32)
            + ga_ref[...].astype(jnp.float32) * p_ref[g].astype(jnp.float32)
        ).astype(jnp.bfloat16)

    all_reduce(x1_ref, final1, sem_base=0, recv_base=0)

    for c in range(N_DEV):
        x1c = x1_ref[c].astype(jnp.float32)
        mu = jnp.mean(x1c, axis=-1, keepdims=True)
        var = jnp.mean((x1c - mu) * (x1c - mu), axis=-1, keepdims=True)
        xm = (x1c - mu) * lax.rsqrt(var + EPS)
        xm = xm * (1.0 + sm_ref[...].astype(jnp.float32)) + shm_ref[...].astype(jnp.float32)
        h1 = jnp.dot(xm.astype(jnp.bfloat16), w1_ref[...],
                     preferred_element_type=jnp.float32)
        h1 = h1 * jax.nn.sigmoid(h1)
        p_ref[c] = jnp.dot(h1.astype(jnp.bfloat16), w2_ref[...],
                           preferred_element_type=jnp.float32).astype(jnp.bfloat16)

    def final2(g):
        out_ref[g] = (
            x1_ref[g].astype(jnp.float32)
            + gm_ref[...].astype(jnp.float32) * p_ref[g].astype(jnp.float32)
        ).astype(jnp.bfloat16)

    all_reduce(out_ref, final2, sem_base=6, recv_base=3)


def _fused(o8, wo8, x0c, ga, sm, shm, gm, w1, w2):
    return pl.pallas_call(
        _fused_body,
        out_shape=jax.ShapeDtypeStruct((N_DEV, CHUNK, D), jnp.bfloat16),
        in_specs=[pl.BlockSpec(memory_space=pltpu.VMEM)] * 9,
        out_specs=pl.BlockSpec(memory_space=pltpu.VMEM),
        scratch_shapes=[
            pltpu.VMEM((N_DEV, CHUNK, D), jnp.bfloat16),
            pltpu.VMEM((N_DEV, CHUNK, D), jnp.bfloat16),
            pltpu.VMEM((6, CHUNK, D), jnp.bfloat16),
            pltpu.SemaphoreType.DMA((12,)),
            pltpu.SemaphoreType.DMA((12,)),
        ],
        compiler_params=pltpu.CompilerParams(
            collective_id=0, vmem_limit_bytes=100 * 1024 * 1024,
        ),
    )(o8, wo8, x0c, ga, sm, shm, gm, w1, w2)


def kernel(x, Wq, Wk, Wv, Wo, t_emb, W_mod, W_ff1, W_ff2):
    f32 = jnp.float32
    bf16 = jnp.bfloat16

    x0 = x.reshape(S, D)
    mod = jnp.dot(t_emb, W_mod)
    sa, sha, ga, sm, shm, gm = jnp.split(mod, 6, axis=-1)

    x0c = x0.astype(bf16).reshape(N_DEV, CHUNK, D)
    q8, k8, v8 = _qkv(
        x0c,
        Wq.astype(bf16),
        Wk.astype(bf16),
        Wv.astype(bf16),
        sa.astype(bf16),
        sha.astype(bf16),
    )
    o8 = _attn(q8, k8, v8)

    res = _fused(
        o8,
        Wo.astype(bf16).reshape(8, DH, D),
        x0c,
        ga.astype(bf16),
        sm.astype(bf16),
        shm.astype(bf16),
        gm.astype(bf16),
        W_ff1.astype(bf16),
        W_ff2.astype(bf16),
    )
    return res.reshape(1, S, D).astype(f32)


# device time: 452538 ns/iter; 1.8234x vs baseline; 1.8151x over previous
import jax
import jax.numpy as jnp
from jax import lax
from jax.experimental import pallas as pl
from jax.experimental.pallas import tpu as pltpu

N_DEV = 4
S = 4096
D = 1024
CHUNK = S // N_DEV
DH = 128
SCALE = 0.08838834764831843
EPS = 1e-5


def _qkv_body(
    x0_ref, wq_ref, wk_ref, wv_ref, sa_ref, sha_ref,
    q8_ref, k8_ref, v8_ref,
):
    HL = 8
    for c in range(N_DEV):
        xc = x0_ref[c].astype(jnp.float32)
        mu = jnp.mean(xc, axis=-1, keepdims=True)
        var = jnp.mean((xc - mu) * (xc - mu), axis=-1, keepdims=True)
        xa = (xc - mu) * lax.rsqrt(var + EPS)
        xa = xa * (1.0 + sa_ref[...].astype(jnp.float32)) + sha_ref[...].astype(jnp.float32)
        xab = xa.astype(jnp.bfloat16)
        qc = jnp.dot(xab, wq_ref[...], preferred_element_type=jnp.float32).astype(jnp.bfloat16)
        kc = jnp.dot(xab, wk_ref[...], preferred_element_type=jnp.float32).astype(jnp.bfloat16)
        vc = jnp.dot(xab, wv_ref[...], preferred_element_type=jnp.float32).astype(jnp.bfloat16)
        for h in range(HL):
            q8_ref[h, pl.ds(c * CHUNK, CHUNK), :] = qc[:, h * DH:(h + 1) * DH]
            k8_ref[h, pl.ds(c * CHUNK, CHUNK), :] = kc[:, h * DH:(h + 1) * DH]
            v8_ref[h, pl.ds(c * CHUNK, CHUNK), :] = vc[:, h * DH:(h + 1) * DH]


def _qkv(x0c, wq, wk, wv, sa, sha):
    shp = jax.ShapeDtypeStruct((8, S, DH), jnp.bfloat16)
    return pl.pallas_call(
        _qkv_body,
        out_shape=(shp, shp, shp),
        in_specs=[pl.BlockSpec(memory_space=pltpu.VMEM)] * 6,
        out_specs=(pl.BlockSpec(memory_space=pltpu.VMEM),) * 3,
        compiler_params=pltpu.CompilerParams(
            vmem_limit_bytes=100 * 1024 * 1024,
        ),
    )(x0c, wq, wk, wv, sa, sha)


def _attn_body(q_ref, k_ref, v_ref, o_ref):
    s = lax.dot_general(
        q_ref[0], k_ref[0], (((1,), (1,)), ((), ())),
        preferred_element_type=jnp.float32,
    ) * SCALE
    p = jnp.exp(s)
    l_ = jnp.sum(p, axis=-1, keepdims=True)
    o = jnp.dot(p.astype(jnp.bfloat16), v_ref[0],
                preferred_element_type=jnp.float32)
    o_ref[0] = (o / l_).astype(jnp.bfloat16)


def _attn(q8, k8, v8):
    return pl.pallas_call(
        _attn_body,
        grid=(8, N_DEV),
        out_shape=jax.ShapeDtypeStruct((8, S, DH), jnp.bfloat16),
        in_specs=[
            pl.BlockSpec((1, CHUNK, DH), lambda h, c: (h, c, 0)),
            pl.BlockSpec((1, S, DH), lambda h, c: (h, 0, 0)),
            pl.BlockSpec((1, S, DH), lambda h, c: (h, 0, 0)),
        ],
        out_specs=pl.BlockSpec((1, CHUNK, DH), lambda h, c: (h, c, 0)),
        compiler_params=pltpu.CompilerParams(
            vmem_limit_bytes=100 * 1024 * 1024,
        ),
    )(q8, k8, v8)


def _fused_body(
    o8_ref, wo_ref, x0_ref, ga_ref, sm_ref, shm_ref, gm_ref, w1_ref, w2_ref,
    out_ref, p_ref, x1_ref, recvR_ref, recvL_ref, send_sems, recv_sems,
):
    my = lax.axis_index("i")
    left = lax.rem(my + N_DEV - 1, N_DEV)
    right = lax.rem(my + 1, N_DEV)

    barrier = pltpu.get_barrier_semaphore()
    pl.semaphore_signal(barrier, inc=1, device_id=(left,),
                        device_id_type=pl.DeviceIdType.MESH)
    pl.semaphore_signal(barrier, inc=1, device_id=(right,),
                        device_id_type=pl.DeviceIdType.MESH)
    pl.semaphore_wait(barrier, 2)

    HALF = CHUNK // 2

    def all_reduce(dst_ref, make_final, sem_base, recv_base):
        for h in range(N_DEV - 1):
            sR = lax.rem(my + ((-h) % N_DEV), N_DEV)
            rR = lax.rem(my + ((-h - 1) % N_DEV), N_DEV)
            sL = lax.rem(my + h, N_DEV)
            rL = lax.rem(my + h + 1, N_DEV)
            rdmaR = pltpu.make_async_remote_copy(
                src_ref=p_ref.at[sR, pl.ds(0, HALF), :],
                dst_ref=recvR_ref.at[recv_base + h],
                send_sem=send_sems.at[sem_base + h],
                recv_sem=recv_sems.at[sem_base + h],
                device_id=(right,),
                device_id_type=pl.DeviceIdType.MESH,
            )
            rdmaL = pltpu.make_async_remote_copy(
                src_ref=p_ref.at[sL, pl.ds(HALF, HALF), :],
                dst_ref=recvL_ref.at[recv_base + h],
                send_sem=send_sems.at[sem_base + 3 + h],
                recv_sem=recv_sems.at[sem_base + 3 + h],
                device_id=(left,),
                device_id_type=pl.DeviceIdType.MESH,
            )
            rdmaR.start()
            rdmaL.start()
            rdmaR.wait()
            rdmaL.wait()
            p_ref[rR, pl.ds(0, HALF), :] = (
                p_ref[rR, pl.ds(0, HALF), :] + recvR_ref[recv_base + h]
            )
            p_ref[rL, pl.ds(HALF, HALF), :] = (
                p_ref[rL, pl.ds(HALF, HALF), :] + recvL_ref[recv_base + h]
            )

        gR = lax.rem(my + 1, N_DEV)
        gL = lax.rem(my + N_DEV - 1, N_DEV)
        make_final(gR, 0)
        make_final(gL, HALF)

        for h in range(N_DEV - 1):
            cR = lax.rem(my + ((1 - h) % N_DEV), N_DEV)
            cL = lax.rem(my + N_DEV - 1 + h, N_DEV)
            rdmaR = pltpu.make_async_remote_copy(
                src_ref=dst_ref.at[cR, pl.ds(0, HALF), :],
                dst_ref=dst_ref.at[cR, pl.ds(0, HALF), :],
                send_sem=send_sems.at[sem_base + 6 + h],
                recv_sem=recv_sems.at[sem_base + 6 + h],
                device_id=(right,),
                device_id_type=pl.DeviceIdType.MESH,
            )
            rdmaL = pltpu.make_async_remote_copy(
                src_ref=dst_ref.at[cL, pl.ds(HALF, HALF), :],
                dst_ref=dst_ref.at[cL, pl.ds(HALF, HALF), :],
                send_sem=send_sems.at[sem_base + 9 + h],
                recv_sem=recv_sems.at[sem_base + 9 + h],
                device_id=(left,),
                device_id_type=pl.DeviceIdType.MESH,
            )
            rdmaR.start()
            rdmaL.start()
            rdmaR.wait()
            rdmaL.wait()

    for c in range(N_DEV):
        acc = jnp.dot(o8_ref[0, pl.ds(c * CHUNK, CHUNK), :], wo_ref[0],
                      preferred_element_type=jnp.float32)
        for h in range(1, 8):
            acc = acc + jnp.dot(o8_ref[h, pl.ds(c * CHUNK, CHUNK), :], wo_ref[h],
                                preferred_element_type=jnp.float32)
        p_ref[c] = acc.astype(jnp.bfloat16)

    def final1(g, off):
        x1_ref[g, pl.ds(off, HALF), :] = (
            x0_ref[g, pl.ds(off, HALF), :].astype(jnp.float32)
            + ga_ref[...].astype(jnp.float32)
            * p_ref[g, pl.ds(off, HALF), :].astype(jnp.float32)
        ).astype(jnp.bfloat16)

    all_reduce(x1_ref, final1, sem_base=0, recv_base=0)

    for c in range(N_DEV):
        x1c = x1_ref[c].astype(jnp.float32)
        mu = jnp.mean(x1c, axis=-1, keepdims=True)
        var = jnp.mean((x1c - mu) * (x1c - mu), axis=-1, keepdims=True)
        xm = (x1c - mu) * lax.rsqrt(var + EPS)
        xm = xm * (1.0 + sm_ref[...].astype(jnp.float32)) + shm_ref[...].astype(jnp.float32)
        h1 = jnp.dot(xm.astype(jnp.bfloat16), w1_ref[...],
                     preferred_element_type=jnp.float32)
        h1 = h1 * jax.nn.sigmoid(h1)
        p_ref[c] = jnp.dot(h1.astype(jnp.bfloat16), w2_ref[...],
                           preferred_element_type=jnp.float32).astype(jnp.bfloat16)

    def final2(g, off):
        out_ref[g, pl.ds(off, HALF), :] = (
            x1_ref[g, pl.ds(off, HALF), :].astype(jnp.float32)
            + gm_ref[...].astype(jnp.float32)
            * p_ref[g, pl.ds(off, HALF), :].astype(jnp.float32)
        ).astype(jnp.bfloat16)

    all_reduce(out_ref, final2, sem_base=12, recv_base=3)


def _fused(o8, wo8, x0c, ga, sm, shm, gm, w1, w2):
    return pl.pallas_call(
        _fused_body,
        out_shape=jax.ShapeDtypeStruct((N_DEV, CHUNK, D), jnp.bfloat16),
        in_specs=[pl.BlockSpec(memory_space=pltpu.VMEM)] * 9,
        out_specs=pl.BlockSpec(memory_space=pltpu.VMEM),
        scratch_shapes=[
            pltpu.VMEM((N_DEV, CHUNK, D), jnp.bfloat16),
            pltpu.VMEM((N_DEV, CHUNK, D), jnp.bfloat16),
            pltpu.VMEM((6, CHUNK // 2, D), jnp.bfloat16),
            pltpu.VMEM((6, CHUNK // 2, D), jnp.bfloat16),
            pltpu.SemaphoreType.DMA((24,)),
            pltpu.SemaphoreType.DMA((24,)),
        ],
        compiler_params=pltpu.CompilerParams(
            collective_id=0, vmem_limit_bytes=100 * 1024 * 1024,
        ),
    )(o8, wo8, x0c, ga, sm, shm, gm, w1, w2)


def kernel(x, Wq, Wk, Wv, Wo, t_emb, W_mod, W_ff1, W_ff2):
    f32 = jnp.float32
    bf16 = jnp.bfloat16

    x0 = x.reshape(S, D)
    mod = jnp.dot(t_emb, W_mod)
    sa, sha, ga, sm, shm, gm = jnp.split(mod, 6, axis=-1)

    x0c = x0.astype(bf16).reshape(N_DEV, CHUNK, D)
    q8, k8, v8 = _qkv(
        x0c,
        Wq.astype(bf16),
        Wk.astype(bf16),
        Wv.astype(bf16),
        sa.astype(bf16),
        sha.astype(bf16),
    )
    o8 = _attn(q8, k8, v8)

    res = _fused(
        o8,
        Wo.astype(bf16).reshape(8, DH, D),
        x0c,
        ga.astype(bf16),
        sm.astype(bf16),
        shm.astype(bf16),
        gm.astype(bf16),
        W_ff1.astype(bf16),
        W_ff2.astype(bf16),
    )
    return res.reshape(1, S, D).astype(f32)


# device time: 425599 ns/iter; 1.9388x vs baseline; 1.0633x over previous
import jax
import jax.numpy as jnp
from jax import lax
from jax.experimental import pallas as pl
from jax.experimental.pallas import tpu as pltpu

N_DEV = 4
S = 4096
D = 1024
CHUNK = S // N_DEV
DH = 128
SCALE = 0.08838834764831843
EPS = 1e-5


def _qkv_body(
    x0_ref, wq_ref, wk_ref, wv_ref, sa_ref, sha_ref,
    q8_ref, k8_ref, v8_ref,
):
    HL = 8
    for c in range(N_DEV):
        xc = x0_ref[c].astype(jnp.float32)
        mu = jnp.mean(xc, axis=-1, keepdims=True)
        var = jnp.mean((xc - mu) * (xc - mu), axis=-1, keepdims=True)
        xa = (xc - mu) * lax.rsqrt(var + EPS)
        xa = xa * (1.0 + sa_ref[...].astype(jnp.float32)) + sha_ref[...].astype(jnp.float32)
        xab = xa.astype(jnp.bfloat16)
        qc = jnp.dot(xab, wq_ref[...], preferred_element_type=jnp.float32).astype(jnp.bfloat16)
        kc = jnp.dot(xab, wk_ref[...], preferred_element_type=jnp.float32).astype(jnp.bfloat16)
        vc = jnp.dot(xab, wv_ref[...], preferred_element_type=jnp.float32).astype(jnp.bfloat16)
        for h in range(HL):
            q8_ref[h, pl.ds(c * CHUNK, CHUNK), :] = qc[:, h * DH:(h + 1) * DH]
            k8_ref[h, pl.ds(c * CHUNK, CHUNK), :] = kc[:, h * DH:(h + 1) * DH]
            v8_ref[h, pl.ds(c * CHUNK, CHUNK), :] = vc[:, h * DH:(h + 1) * DH]


def _qkv(x0c, wq, wk, wv, sa, sha):
    shp = jax.ShapeDtypeStruct((8, S, DH), jnp.bfloat16)
    return pl.pallas_call(
        _qkv_body,
        out_shape=(shp, shp, shp),
        in_specs=[pl.BlockSpec(memory_space=pltpu.VMEM)] * 6,
        out_specs=(pl.BlockSpec(memory_space=pltpu.VMEM),) * 3,
        compiler_params=pltpu.CompilerParams(
            vmem_limit_bytes=100 * 1024 * 1024,
        ),
    )(x0c, wq, wk, wv, sa, sha)


def _attn_body(q_ref, k_ref, v_ref, o_ref):
    s = lax.dot_general(
        q_ref[0], k_ref[0], (((1,), (1,)), ((), ())),
        preferred_element_type=jnp.float32,
    ) * SCALE
    p = jnp.exp(s)
    l_ = jnp.sum(p, axis=-1, keepdims=True)
    o = jnp.dot(p.astype(jnp.bfloat16), v_ref[0],
                preferred_element_type=jnp.float32)
    o_ref[0] = (o / l_).astype(jnp.bfloat16)


def _attn(q8, k8, v8):
    return pl.pallas_call(
        _attn_body,
        grid=(8, N_DEV),
        out_shape=jax.ShapeDtypeStruct((8, S, DH), jnp.bfloat16),
        in_specs=[
            pl.BlockSpec((1, CHUNK, DH), lambda h, c: (h, c, 0)),
            pl.BlockSpec((1, S, DH), lambda h, c: (h, 0, 0)),
            pl.BlockSpec((1, S, DH), lambda h, c: (h, 0, 0)),
        ],
        out_specs=pl.BlockSpec((1, CHUNK, DH), lambda h, c: (h, c, 0)),
        compiler_params=pltpu.CompilerParams(
            vmem_limit_bytes=100 * 1024 * 1024,
        ),
    )(q8, k8, v8)


def _fused_body(
    o8_ref, wo_ref, x0_ref, ga_ref, sm_ref, shm_ref, gm_ref, w1_ref, w2_ref,
    out_ref, p_ref, x1_ref, recvR_ref, recvL_ref, send_sems, recv_sems,
):
    my = lax.axis_index("i")
    left = lax.rem(my + N_DEV - 1, N_DEV)
    right = lax.rem(my + 1, N_DEV)

    barrier = pltpu.get_barrier_semaphore()
    pl.semaphore_signal(barrier, inc=1, device_id=(left,),
                        device_id_type=pl.DeviceIdType.MESH)
    pl.semaphore_signal(barrier, inc=1, device_id=(right,),
                        device_id_type=pl.DeviceIdType.MESH)
    pl.semaphore_wait(barrier, 2)

    HALF = CHUNK // 2


    def mk_rs(h, sem_base, recv_base):
        sR = lax.rem(my + ((-h) % N_DEV), N_DEV)
        sL = lax.rem(my + h, N_DEV)
        rdmaR = pltpu.make_async_remote_copy(
            src_ref=p_ref.at[sR, pl.ds(0, HALF), :],
            dst_ref=recvR_ref.at[recv_base + h],
            send_sem=send_sems.at[sem_base + h],
            recv_sem=recv_sems.at[sem_base + h],
            device_id=(right,),
            device_id_type=pl.DeviceIdType.MESH,
        )
        rdmaL = pltpu.make_async_remote_copy(
            src_ref=p_ref.at[sL, pl.ds(HALF, HALF), :],
            dst_ref=recvL_ref.at[recv_base + h],
            send_sem=send_sems.at[sem_base + 3 + h],
            recv_sem=recv_sems.at[sem_base + 3 + h],
            device_id=(left,),
            device_id_type=pl.DeviceIdType.MESH,
        )
        rdmaR.start()
        rdmaL.start()
        return rdmaR, rdmaL

    def rs_finish(pair, h, recv_base):
        pair[0].wait()
        pair[1].wait()
        rR = lax.rem(my + ((-h - 1) % N_DEV), N_DEV)
        rL = lax.rem(my + h + 1, N_DEV)
        p_ref[rR, pl.ds(0, HALF), :] = (
            p_ref[rR, pl.ds(0, HALF), :] + recvR_ref[recv_base + h]
        )
        p_ref[rL, pl.ds(HALF, HALF), :] = (
            p_ref[rL, pl.ds(HALF, HALF), :] + recvL_ref[recv_base + h]
        )

    def mk_ag(dst_ref, h, sem_base):
        cR = lax.rem(my + ((1 - h) % N_DEV), N_DEV)
        cL = lax.rem(my + N_DEV - 1 + h, N_DEV)
        rdmaR = pltpu.make_async_remote_copy(
            src_ref=dst_ref.at[cR, pl.ds(0, HALF), :],
            dst_ref=dst_ref.at[cR, pl.ds(0, HALF), :],
            send_sem=send_sems.at[sem_base + 6 + h],
            recv_sem=recv_sems.at[sem_base + 6 + h],
            device_id=(right,),
            device_id_type=pl.DeviceIdType.MESH,
        )
        rdmaL = pltpu.make_async_remote_copy(
            src_ref=dst_ref.at[cL, pl.ds(HALF, HALF), :],
            dst_ref=dst_ref.at[cL, pl.ds(HALF, HALF), :],
            send_sem=send_sems.at[sem_base + 9 + h],
            recv_sem=recv_sems.at[sem_base + 9 + h],
            device_id=(left,),
            device_id_type=pl.DeviceIdType.MESH,
        )
        rdmaR.start()
        rdmaL.start()
        return rdmaR, rdmaL

    def ag_wait(pair):
        pair[0].wait()
        pair[1].wait()

    def compute_p1(c):
        acc = jnp.dot(o8_ref[0, pl.ds(c * CHUNK, CHUNK), :], wo_ref[0],
                      preferred_element_type=jnp.float32)
        for h in range(1, 8):
            acc = acc + jnp.dot(o8_ref[h, pl.ds(c * CHUNK, CHUNK), :], wo_ref[h],
                                preferred_element_type=jnp.float32)
        p_ref[c] = acc.astype(jnp.bfloat16)

    def final1(g, off):
        x1_ref[g, pl.ds(off, HALF), :] = (
            x0_ref[g, pl.ds(off, HALF), :].astype(jnp.float32)
            + ga_ref[...].astype(jnp.float32)
            * p_ref[g, pl.ds(off, HALF), :].astype(jnp.float32)
        ).astype(jnp.bfloat16)

    def ff(c):
        x1c = x1_ref[c].astype(jnp.float32)
        mu = jnp.mean(x1c, axis=-1, keepdims=True)
        var = jnp.mean((x1c - mu) * (x1c - mu), axis=-1, keepdims=True)
        xm = (x1c - mu) * lax.rsqrt(var + EPS)
        xm = xm * (1.0 + sm_ref[...].astype(jnp.float32)) + shm_ref[...].astype(jnp.float32)
        h1 = jnp.dot(xm.astype(jnp.bfloat16), w1_ref[...],
                     preferred_element_type=jnp.float32)
        h1 = h1 * jax.nn.sigmoid(h1)
        p_ref[c] = jnp.dot(h1.astype(jnp.bfloat16), w2_ref[...],
                           preferred_element_type=jnp.float32).astype(jnp.bfloat16)

    def final2(g, off):
        out_ref[g, pl.ds(off, HALF), :] = (
            x1_ref[g, pl.ds(off, HALF), :].astype(jnp.float32)
            + gm_ref[...].astype(jnp.float32)
            * p_ref[g, pl.ds(off, HALF), :].astype(jnp.float32)
        ).astype(jnp.bfloat16)

    compute_p1(my)
    rs0 = mk_rs(0, 0, 0)
    for j in range(1, N_DEV):
        compute_p1(lax.rem(my + j, N_DEV))
    rs_finish(rs0, 0, 0)
    for h in (1, 2):
        rs_finish(mk_rs(h, 0, 0), h, 0)

    gR = lax.rem(my + 1, N_DEV)
    gL = lax.rem(my + N_DEV - 1, N_DEV)
    final1(gR, 0)
    final1(gL, HALF)

    ag_wait(mk_ag(x1_ref, 0, 0))
    a1 = mk_ag(x1_ref, 1, 0)
    ff(my)
    rs20 = mk_rs(0, 12, 0)
    ag_wait(a1)
    a2 = mk_ag(x1_ref, 2, 0)
    ff(gR)
    ff(gL)
    ag_wait(a2)
    ff(lax.rem(my + 2, N_DEV))
    rs_finish(rs20, 0, 0)
    for h in (1, 2):
        rs_finish(mk_rs(h, 12, 0), h, 0)

    final2(gR, 0)
    final2(gL, HALF)
    for h in range(N_DEV - 1):
        ag_wait(mk_ag(out_ref, h, 12))


def _fused(o8, wo8, x0c, ga, sm, shm, gm, w1, w2):
    return pl.pallas_call(
        _fused_body,
        out_shape=jax.ShapeDtypeStruct((N_DEV, CHUNK, D), jnp.bfloat16),
        in_specs=[pl.BlockSpec(memory_space=pltpu.VMEM)] * 9,
        out_specs=pl.BlockSpec(memory_space=pltpu.VMEM),
        scratch_shapes=[
            pltpu.VMEM((N_DEV, CHUNK, D), jnp.bfloat16),
            pltpu.VMEM((N_DEV, CHUNK, D), jnp.bfloat16),
            pltpu.VMEM((3, CHUNK // 2, D), jnp.bfloat16),
            pltpu.VMEM((3, CHUNK // 2, D), jnp.bfloat16),
            pltpu.SemaphoreType.DMA((24,)),
            pltpu.SemaphoreType.DMA((24,)),
        ],
        compiler_params=pltpu.CompilerParams(
            collective_id=0, vmem_limit_bytes=100 * 1024 * 1024,
        ),
    )(o8, wo8, x0c, ga, sm, shm, gm, w1, w2)


def kernel(x, Wq, Wk, Wv, Wo, t_emb, W_mod, W_ff1, W_ff2):
    f32 = jnp.float32
    bf16 = jnp.bfloat16

    x0 = x.reshape(S, D)
    mod = jnp.dot(t_emb, W_mod)
    sa, sha, ga, sm, shm, gm = jnp.split(mod, 6, axis=-1)

    x0c = x0.astype(bf16).reshape(N_DEV, CHUNK, D)
    q8, k8, v8 = _qkv(
        x0c,
        Wq.astype(bf16),
        Wk.astype(bf16),
        Wv.astype(bf16),
        sa.astype(bf16),
        sha.astype(bf16),
    )
    o8 = _attn(q8, k8, v8)

    res = _fused(
        o8,
        Wo.astype(bf16).reshape(8, DH, D),
        x0c,
        ga.astype(bf16),
        sm.astype(bf16),
        shm.astype(bf16),
        gm.astype(bf16),
        W_ff1.astype(bf16),
        W_ff2.astype(bf16),
    )
    return res.reshape(1, S, D).astype(f32)
